# baseline (speedup 1.0000x reference)
import functools

import jax
import jax.numpy as jnp
from jax import lax
from jax.experimental import pallas as pl
from jax.experimental.pallas import tpu as pltpu
from jax.experimental.pallas import tpu_sc as plsc

N = 10000
E = 320000
R = 1000
EH = 128
RH = 128

NC = 2
NS = 16
L = 16

NPT = N // NS
EBLK = 1600
NBLK = E // EBLK
VPB = EBLK // L
GRP = 4
BATCH = GRP * L
STG = 1024
ABLKS = 25
ACAP = ABLKS * STG


def _proj_body(xe_ref, xr_ref, we_ref, wr_ref, se_ref, sr_ref):
    se_ref[...] = jnp.dot(xe_ref[...], we_ref[...],
                          preferred_element_type=jnp.float32)
    sr_ref[...] = jnp.dot(xr_ref[...], wr_ref[...],
                          preferred_element_type=jnp.float32)


def _projections(x_e, x_r, w_e, w_r):
    return pl.pallas_call(
        _proj_body,
        out_shape=(jax.ShapeDtypeStruct((N, 128), jnp.float32),
                   jax.ShapeDtypeStruct((R, 128), jnp.float32)),
    )(x_e, x_r, w_e, w_r)


_mesh = plsc.VectorSubcoreMesh(core_axis_name="c", subcore_axis_name="s")


@functools.partial(
    pl.kernel,
    out_type=(jax.ShapeDtypeStruct((NC * NS, NPT, RH), jnp.float32),
              jax.ShapeDtypeStruct((NC * NS, ABLKS, STG), jnp.int32),
              jax.ShapeDtypeStruct((NC * NS, ABLKS, STG), jnp.float32)),
    mesh=_mesh,
    scratch_types=[
        pltpu.VMEM((N,), jnp.float32),
        pltpu.VMEM((R,), jnp.float32),
        pltpu.VMEM((2 * EBLK,), jnp.int32),
        pltpu.VMEM((2 * EBLK,), jnp.int32),
        pltpu.VMEM((NPT + 16,), jnp.float32),
        pltpu.VMEM((NPT, RH), jnp.float32),
        pltpu.VMEM((STG + L,), jnp.int32),
        pltpu.VMEM((STG + L,), jnp.float32),
        pltpu.VMEM((STG,), jnp.int32),
        pltpu.VMEM((STG,), jnp.float32),
        pltpu.VMEM((2, BATCH), jnp.int32),
        pltpu.VMEM((2, BATCH), jnp.int32),
        pltpu.VMEM((2, BATCH), jnp.float32),
        pltpu.VMEM((2, BATCH, RH), jnp.float32),
        pltpu.SemaphoreType.DMA,
        pltpu.SemaphoreType.DMA,
    ],
    compiler_params=pltpu.CompilerParams(needs_layout_passes=False),
)
def _sc_gat(s_ht, s_r, eidx, rel, x_r, out, apk, aex,
            s_v, sr_v, idx_v, rel_v, den_v, acc_v,
            spk_v, sex_v, rpk_v, rex_v,
            br_v, bi_v, ba_v, rows_v, sem, sem_e):
    c = lax.axis_index("c")
    s = lax.axis_index("s")
    base = s * NPT
    wid = c * NS + s

    pltpu.sync_copy(s_ht.at[pl.ds(c * N, N)], s_v)
    pltpu.sync_copy(s_r, sr_v)

    zf = jnp.zeros((L,), jnp.float32)

    def zden(i, carry):
        den_v[pl.ds(i * L, L)] = zf
        return carry
    lax.fori_loop(0, (NPT + 16) // L, zden, 0)

    def zacc(i, carry):
        for j in range(RH // L):
            acc_v[i, pl.ds(j * L, L)] = zf
        return carry
    lax.fori_loop(0, NPT, zacc, 0)

    def issue_edges(b, blk):
        pltpu.async_copy(eidx.at[pl.ds(c * E + b * EBLK, EBLK)],
                         idx_v.at[pl.ds(blk * EBLK, EBLK)], sem_e)
        pltpu.async_copy(rel.at[pl.ds(b * EBLK, EBLK)],
                         rel_v.at[pl.ds(blk * EBLK, EBLK)], sem_e)

    def wait_edges(blk):
        pltpu.make_async_copy(rel.at[pl.ds(0, EBLK)],
                              idx_v.at[pl.ds(blk * EBLK, EBLK)], sem_e).wait()
        pltpu.make_async_copy(rel.at[pl.ds(0, EBLK)],
                              rel_v.at[pl.ds(blk * EBLK, EBLK)], sem_e).wait()

    def p1_step(blk, i, cnt):
        nd = idx_v[pl.ds(blk * EBLK + i * L, L)]
        rl = rel_v[pl.ds(blk * EBLK + i * L, L)]
        il = nd - base
        m = (il >= 0) & (il < NPT)
        ilc = jnp.clip(il, 0, NPT - 1)
        sh = plsc.load_gather(s_v, [nd])
        sr = plsc.load_gather(sr_v, [rl])
        z = sh + sr
        ex = jnp.exp(jnp.maximum(z, 0.01 * z))
        plsc.addupdate_scatter(den_v, [ilc], ex, mask=m)
        pk = ilc | (rl << 10)
        scnt = lax.rem(cnt, STG)
        fb = cnt // STG
        plsc.store_compressed(spk_v.at[pl.ds(scnt, L)], pk, mask=m)
        plsc.store_compressed(sex_v.at[pl.ds(scnt, L)], ex, mask=m)
        cnt = cnt + plsc.all_reduce_population_count(m)[0]
        scnt2 = lax.rem(cnt, STG)

        @pl.when((cnt // STG > fb) & (fb < ABLKS))
        def _():
            pltpu.sync_copy(spk_v.at[pl.ds(0, STG)], apk.at[wid, fb])
            pltpu.sync_copy(sex_v.at[pl.ds(0, STG)], aex.at[wid, fb])
            rem_pk = spk_v[pl.ds(STG, L)]
            rem_ex = sex_v[pl.ds(STG, L)]
            rmask = lax.iota(jnp.int32, L) < scnt2
            plsc.store_compressed(spk_v.at[pl.ds(0, L)], rem_pk, mask=rmask)
            plsc.store_compressed(sex_v.at[pl.ds(0, L)], rem_ex, mask=rmask)
        return jnp.minimum(cnt, ACAP - 1)

    issue_edges(0, 0)

    def p1_block(b, cnt):
        blk = lax.rem(b, 2)
        wait_edges(blk)

        @pl.when(b + 1 < NBLK)
        def _():
            issue_edges(b + 1, 1 - blk)

        def step(i, cnt):
            cnt = p1_step(blk, i * 2, cnt)
            return p1_step(blk, i * 2 + 1, cnt)
        return lax.fori_loop(0, VPB // 2, step, cnt)
    cnt = lax.fori_loop(0, NBLK, p1_block, 0)

    lastfb = jnp.minimum(cnt // STG, ABLKS - 1)
    pltpu.sync_copy(spk_v.at[pl.ds(0, STG)], apk.at[wid, lastfb])
    pltpu.sync_copy(sex_v.at[pl.ds(0, STG)], aex.at[wid, lastfb])

    def issue_flush(p):
        pltpu.async_copy(x_r.at[br_v.at[p]], rows_v.at[p], sem)

    def wait_rows(p):
        pltpu.make_async_copy(x_r.at[pl.ds(0, BATCH)], rows_v.at[p],
                              sem).wait()

    def accum(p):
        def fgroup(g, carry):
            avec = ba_v[p, pl.ds(g * L, L)]
            ivec = bi_v[p, pl.ds(g * L, L)]
            for j in range(L):
                av = jnp.full((L,), avec[j])
                ii = ivec[j]
                for q in range(RH // L):
                    plsc.addupdate(acc_v.at[ii, pl.ds(q * L, L)],
                                   av * rows_v[p, g * L + j, pl.ds(q * L, L)])
            return carry
        lax.fori_loop(0, GRP, fgroup, 0)

    nblk2 = (cnt + STG - 1) // STG

    def p2_block(fb, carry):
        par, pend = carry
        pltpu.sync_copy(apk.at[wid, fb], rpk_v)
        pltpu.sync_copy(aex.at[wid, fb], rex_v)
        gbase0 = fb * STG

        def p2_batch(kb, carry):
            par, pend = carry
            for u in range(GRP):
                off = kb * BATCH + u * L
                pk = rpk_v[pl.ds(off, L)]
                exv = rex_v[pl.ds(off, L)]
                valid = (gbase0 + off + lax.iota(jnp.int32, L)) < cnt
                ilc = jnp.minimum(pk & 0x3FF, NPT - 1)
                rl = jnp.minimum((pk >> 10) & 0x3FF, R - 1)
                dv = plsc.load_gather(den_v, [ilc])
                a = jnp.where(valid, exv / (dv + 1e-16), 0.0)
                br_v[par, pl.ds(u * L, L)] = rl
                bi_v[par, pl.ds(u * L, L)] = ilc
                ba_v[par, pl.ds(u * L, L)] = a

            @pl.when(pend == 1)
            def _():
                wait_rows(1 - par)
                accum(1 - par)
            issue_flush(par)
            return (1 - par, pend * 0 + 1)
        return lax.fori_loop(0, STG // BATCH, p2_batch, (par, pend))
    par, pend = lax.fori_loop(0, nblk2, p2_block, (0, 0))

    @pl.when(pend == 1)
    def _():
        wait_rows(1 - par)
        accum(1 - par)

    pltpu.sync_copy(acc_v, out.at[wid])


def kernel(x_e, x_r, edge_index, rel, line_graph_index, line_graph_val,
           w_h, w_t, w_r):
    del line_graph_index, line_graph_val
    w_e = jnp.zeros((EH, 128), jnp.float32).at[:, 0].set(w_h).at[:, 1].set(w_t)
    w_rp = jnp.zeros((RH, 128), jnp.float32).at[:, 0].set(w_r)
    se, sr = _projections(x_e, x_r, w_e, w_rp)
    s_ht = se[:, :2].T.reshape(-1)
    s_r1 = sr[:, 0]
    eflat = edge_index.reshape(-1)
    o = _sc_gat(s_ht, s_r1, eflat, rel, x_r)[0]
    x_e_h = o[:NS].reshape(N, RH)
    x_e_t = o[NS:].reshape(N, RH)
    return jnp.concatenate([x_e_h, x_e_t], axis=1)


# speedup vs baseline: 1.1041x; 1.1041x over previous
import functools

import jax
import jax.numpy as jnp
from jax import lax
from jax.experimental import pallas as pl
from jax.experimental.pallas import tpu as pltpu
from jax.experimental.pallas import tpu_sc as plsc

N = 10000
E = 320000
R = 1000
EH = 128
RH = 128

NC = 2
NS = 16
L = 16

NPT = N // NS
EBLK = 1600
NBLK = E // EBLK
VPB = EBLK // L
GRP = 4
BATCH = GRP * L
STG = 1024
ABLKS = 25
ACAP = ABLKS * STG


def _proj_body(xe_ref, xr_ref, we_ref, wr_ref, se_ref, sr_ref):
    se_ref[...] = jnp.dot(xe_ref[...], we_ref[...],
                          preferred_element_type=jnp.float32)
    sr_ref[...] = jnp.dot(xr_ref[...], wr_ref[...],
                          preferred_element_type=jnp.float32)


def _projections(x_e, x_r, w_e, w_r):
    return pl.pallas_call(
        _proj_body,
        out_shape=(jax.ShapeDtypeStruct((N, 128), jnp.float32),
                   jax.ShapeDtypeStruct((R, 128), jnp.float32)),
    )(x_e, x_r, w_e, w_r)


_mesh = plsc.VectorSubcoreMesh(core_axis_name="c", subcore_axis_name="s")


@functools.partial(
    pl.kernel,
    out_type=(jax.ShapeDtypeStruct((NC * NS, NPT, RH), jnp.float32),
              jax.ShapeDtypeStruct((NC * NS, ABLKS, STG), jnp.int32),
              jax.ShapeDtypeStruct((NC * NS, ABLKS, STG), jnp.float32)),
    mesh=_mesh,
    scratch_types=[
        pltpu.VMEM((N,), jnp.float32),
        pltpu.VMEM((R,), jnp.float32),
        pltpu.VMEM((2 * EBLK,), jnp.int32),
        pltpu.VMEM((2 * EBLK,), jnp.int32),
        pltpu.VMEM((NPT + 16,), jnp.float32),
        pltpu.VMEM((NPT, RH), jnp.float32),
        pltpu.VMEM((STG + L,), jnp.int32),
        pltpu.VMEM((STG + L,), jnp.float32),
        pltpu.VMEM((STG,), jnp.int32),
        pltpu.VMEM((STG,), jnp.float32),
        pltpu.VMEM((2, BATCH), jnp.int32),
        pltpu.VMEM((2, BATCH), jnp.int32),
        pltpu.VMEM((2, BATCH), jnp.float32),
        pltpu.VMEM((2, BATCH, RH), jnp.float32),
        pltpu.VMEM_SHARED((R, RH), jnp.float32),
        pltpu.SemaphoreType.DMA,
        pltpu.SemaphoreType.DMA,
    ],
    compiler_params=pltpu.CompilerParams(needs_layout_passes=False),
)
def _sc_gat(s_ht, s_r, eidx, rel, x_r, out, apk, aex,
            s_v, sr_v, idx_v, rel_v, den_v, acc_v,
            spk_v, sex_v, rpk_v, rex_v,
            br_v, bi_v, ba_v, rows_v, xr_sh, sem, sem_e):
    c = lax.axis_index("c")
    s = lax.axis_index("s")
    base = s * NPT
    wid = c * NS + s

    @pl.when(s == 0)
    def _():
        pltpu.sync_copy(x_r, xr_sh)

    pltpu.sync_copy(s_ht.at[pl.ds(c * N, N)], s_v)
    pltpu.sync_copy(s_r, sr_v)

    zf = jnp.zeros((L,), jnp.float32)

    def zden(i, carry):
        den_v[pl.ds(i * L, L)] = zf
        return carry
    lax.fori_loop(0, (NPT + 16) // L, zden, 0)

    def zacc(i, carry):
        for j in range(RH // L):
            acc_v[i, pl.ds(j * L, L)] = zf
        return carry
    lax.fori_loop(0, NPT, zacc, 0)

    plsc.subcore_barrier()

    def issue_edges(b, blk):
        pltpu.async_copy(eidx.at[pl.ds(c * E + b * EBLK, EBLK)],
                         idx_v.at[pl.ds(blk * EBLK, EBLK)], sem_e)
        pltpu.async_copy(rel.at[pl.ds(b * EBLK, EBLK)],
                         rel_v.at[pl.ds(blk * EBLK, EBLK)], sem_e)

    def wait_edges(blk):
        pltpu.make_async_copy(rel.at[pl.ds(0, EBLK)],
                              idx_v.at[pl.ds(blk * EBLK, EBLK)], sem_e).wait()
        pltpu.make_async_copy(rel.at[pl.ds(0, EBLK)],
                              rel_v.at[pl.ds(blk * EBLK, EBLK)], sem_e).wait()

    def p1_step(blk, i, cnt):
        nd = idx_v[pl.ds(blk * EBLK + i * L, L)]
        rl = rel_v[pl.ds(blk * EBLK + i * L, L)]
        il = nd - base
        m = (il >= 0) & (il < NPT)
        ilc = jnp.clip(il, 0, NPT - 1)
        sh = plsc.load_gather(s_v, [nd])
        sr = plsc.load_gather(sr_v, [rl])
        z = sh + sr
        ex = jnp.exp(jnp.maximum(z, 0.01 * z))
        plsc.addupdate_scatter(den_v, [ilc], ex, mask=m)
        pk = ilc | (rl << 10)
        scnt = lax.rem(cnt, STG)
        fb = cnt // STG
        plsc.store_compressed(spk_v.at[pl.ds(scnt, L)], pk, mask=m)
        plsc.store_compressed(sex_v.at[pl.ds(scnt, L)], ex, mask=m)
        cnt = cnt + plsc.all_reduce_population_count(m)[0]
        scnt2 = lax.rem(cnt, STG)

        @pl.when((cnt // STG > fb) & (fb < ABLKS))
        def _():
            pltpu.sync_copy(spk_v.at[pl.ds(0, STG)], apk.at[wid, fb])
            pltpu.sync_copy(sex_v.at[pl.ds(0, STG)], aex.at[wid, fb])
            rem_pk = spk_v[pl.ds(STG, L)]
            rem_ex = sex_v[pl.ds(STG, L)]
            rmask = lax.iota(jnp.int32, L) < scnt2
            plsc.store_compressed(spk_v.at[pl.ds(0, L)], rem_pk, mask=rmask)
            plsc.store_compressed(sex_v.at[pl.ds(0, L)], rem_ex, mask=rmask)
        return jnp.minimum(cnt, ACAP - 1)

    issue_edges(0, 0)

    def p1_block(b, cnt):
        blk = lax.rem(b, 2)
        wait_edges(blk)

        @pl.when(b + 1 < NBLK)
        def _():
            issue_edges(b + 1, 1 - blk)

        def step(i, cnt):
            cnt = p1_step(blk, i * 2, cnt)
            return p1_step(blk, i * 2 + 1, cnt)
        return lax.fori_loop(0, VPB // 2, step, cnt)
    cnt = lax.fori_loop(0, NBLK, p1_block, 0)

    lastfb = jnp.minimum(cnt // STG, ABLKS - 1)
    pltpu.sync_copy(spk_v.at[pl.ds(0, STG)], apk.at[wid, lastfb])
    pltpu.sync_copy(sex_v.at[pl.ds(0, STG)], aex.at[wid, lastfb])

    def issue_flush(p):
        pltpu.async_copy(xr_sh.at[br_v.at[p]], rows_v.at[p], sem)

    def wait_rows(p):
        pltpu.make_async_copy(x_r.at[pl.ds(0, BATCH)], rows_v.at[p],
                              sem).wait()

    def accum(p):
        def fgroup(g, carry):
            avec = ba_v[p, pl.ds(g * L, L)]
            ivec = bi_v[p, pl.ds(g * L, L)]
            for j in range(L):
                av = jnp.full((L,), avec[j])
                ii = ivec[j]
                for q in range(RH // L):
                    plsc.addupdate(acc_v.at[ii, pl.ds(q * L, L)],
                                   av * rows_v[p, g * L + j, pl.ds(q * L, L)])
            return carry
        lax.fori_loop(0, GRP, fgroup, 0)

    nblk2 = (cnt + STG - 1) // STG

    def p2_block(fb, carry):
        par, pend = carry
        pltpu.sync_copy(apk.at[wid, fb], rpk_v)
        pltpu.sync_copy(aex.at[wid, fb], rex_v)
        gbase0 = fb * STG

        def p2_batch(kb, carry):
            par, pend = carry
            for u in range(GRP):
                off = kb * BATCH + u * L
                pk = rpk_v[pl.ds(off, L)]
                exv = rex_v[pl.ds(off, L)]
                valid = (gbase0 + off + lax.iota(jnp.int32, L)) < cnt
                ilc = jnp.minimum(pk & 0x3FF, NPT - 1)
                rl = jnp.minimum((pk >> 10) & 0x3FF, R - 1)
                dv = plsc.load_gather(den_v, [ilc])
                a = jnp.where(valid, exv / (dv + 1e-16), 0.0)
                br_v[par, pl.ds(u * L, L)] = rl
                bi_v[par, pl.ds(u * L, L)] = ilc
                ba_v[par, pl.ds(u * L, L)] = a

            @pl.when(pend == 1)
            def _():
                wait_rows(1 - par)
                accum(1 - par)
            issue_flush(par)
            return (1 - par, pend * 0 + 1)
        return lax.fori_loop(0, STG // BATCH, p2_batch, (par, pend))
    par, pend = lax.fori_loop(0, nblk2, p2_block, (0, 0))

    @pl.when(pend == 1)
    def _():
        wait_rows(1 - par)
        accum(1 - par)

    pltpu.sync_copy(acc_v, out.at[wid])


def kernel(x_e, x_r, edge_index, rel, line_graph_index, line_graph_val,
           w_h, w_t, w_r):
    del line_graph_index, line_graph_val
    w_e = jnp.zeros((EH, 128), jnp.float32).at[:, 0].set(w_h).at[:, 1].set(w_t)
    w_rp = jnp.zeros((RH, 128), jnp.float32).at[:, 0].set(w_r)
    se, sr = _projections(x_e, x_r, w_e, w_rp)
    s_ht = se[:, :2].T.reshape(-1)
    s_r1 = sr[:, 0]
    eflat = edge_index.reshape(-1)
    o = _sc_gat(s_ht, s_r1, eflat, rel, x_r)[0]
    x_e_h = o[:NS].reshape(N, RH)
    x_e_t = o[NS:].reshape(N, RH)
    return jnp.concatenate([x_e_h, x_e_t], axis=1)


# speedup vs baseline: 1.3841x; 1.2536x over previous
import functools

import jax
import jax.numpy as jnp
from jax import lax
from jax.experimental import pallas as pl
from jax.experimental.pallas import tpu as pltpu
from jax.experimental.pallas import tpu_sc as plsc

N = 10000
E = 320000
R = 1000
EH = 128
RH = 128

NC = 2
NS = 16
L = 16

NPT = N // NS
EBLK = 1600
NBLK = E // EBLK
VPB = EBLK // L
GRP = 4
BATCH = GRP * L
STG = 1024
ABLKS = 25
ACAP = ABLKS * STG


def _proj_body(xe_ref, xr_ref, we_ref, wr_ref, se_ref, sr_ref):
    se_ref[...] = jnp.dot(xe_ref[...], we_ref[...],
                          preferred_element_type=jnp.float32)
    sr_ref[...] = jnp.dot(xr_ref[...], wr_ref[...],
                          preferred_element_type=jnp.float32)


def _projections(x_e, x_r, w_e, w_r):
    return pl.pallas_call(
        _proj_body,
        out_shape=(jax.ShapeDtypeStruct((N, 128), jnp.float32),
                   jax.ShapeDtypeStruct((R, 128), jnp.float32)),
    )(x_e, x_r, w_e, w_r)


_mesh = plsc.VectorSubcoreMesh(core_axis_name="c", subcore_axis_name="s")


@functools.partial(
    pl.kernel,
    out_type=(jax.ShapeDtypeStruct((NC * NS, NPT, RH), jnp.float32),
              jax.ShapeDtypeStruct((NC * NS, ABLKS, STG), jnp.int32),
              jax.ShapeDtypeStruct((NC * NS, ABLKS, STG), jnp.float32)),
    mesh=_mesh,
    scratch_types=[
        pltpu.VMEM((N,), jnp.float32),
        pltpu.VMEM((R,), jnp.float32),
        pltpu.VMEM((2 * EBLK,), jnp.int32),
        pltpu.VMEM((2 * EBLK,), jnp.int32),
        pltpu.VMEM((NPT + 16,), jnp.float32),
        pltpu.VMEM((NPT, RH), jnp.float32),
        pltpu.VMEM((STG + L,), jnp.int32),
        pltpu.VMEM((STG + L,), jnp.float32),
        pltpu.VMEM((STG,), jnp.int32),
        pltpu.VMEM((STG,), jnp.float32),
        pltpu.VMEM((2, BATCH), jnp.int32),
        pltpu.VMEM((2, BATCH), jnp.int32),
        pltpu.VMEM((2, BATCH), jnp.float32),
        pltpu.VMEM((2, BATCH, RH), jnp.float32),
        pltpu.VMEM_SHARED((R, RH), jnp.float32),
        pltpu.SemaphoreType.DMA,
        pltpu.SemaphoreType.DMA,
    ],
    compiler_params=pltpu.CompilerParams(needs_layout_passes=False),
)
def _sc_gat(s_ht, s_r, eidx, rel, x_r, out, apk, aex,
            s_v, sr_v, idx_v, rel_v, den_v, acc_v,
            spk_v, sex_v, rpk_v, rex_v,
            br_v, bi_v, ba_v, rows_v, xr_sh, sem, sem_e):
    c = lax.axis_index("c")
    s = lax.axis_index("s")
    base = s * NPT
    wid = c * NS + s

    @pl.when(s == 0)
    def _():
        pltpu.sync_copy(x_r, xr_sh)

    pltpu.sync_copy(s_ht.at[pl.ds(c * N, N)], s_v)
    pltpu.sync_copy(s_r, sr_v)

    zf = jnp.zeros((L,), jnp.float32)

    def zden(i, carry):
        den_v[pl.ds(i * L, L)] = zf
        return carry
    lax.fori_loop(0, (NPT + 16) // L, zden, 0)

    def zacc(i, carry):
        for j in range(RH // L):
            acc_v[i, pl.ds(j * L, L)] = zf
        return carry
    lax.fori_loop(0, NPT, zacc, 0)

    plsc.subcore_barrier()

    def issue_edges(b, blk):
        pltpu.async_copy(eidx.at[pl.ds(c * E + b * EBLK, EBLK)],
                         idx_v.at[pl.ds(blk * EBLK, EBLK)], sem_e)
        pltpu.async_copy(rel.at[pl.ds(b * EBLK, EBLK)],
                         rel_v.at[pl.ds(blk * EBLK, EBLK)], sem_e)

    def wait_edges(blk):
        pltpu.make_async_copy(rel.at[pl.ds(0, EBLK)],
                              idx_v.at[pl.ds(blk * EBLK, EBLK)], sem_e).wait()
        pltpu.make_async_copy(rel.at[pl.ds(0, EBLK)],
                              rel_v.at[pl.ds(blk * EBLK, EBLK)], sem_e).wait()

    def edge_vec(blk, i):
        nd = idx_v[pl.ds(blk * EBLK + i * L, L)]
        rl = rel_v[pl.ds(blk * EBLK + i * L, L)]
        il = nd - base
        m = (il >= 0) & (il < NPT)
        ilc = jnp.clip(il, 0, NPT - 1)
        sh = plsc.load_gather(s_v, [nd])
        sr = plsc.load_gather(sr_v, [rl])
        z = sh + sr
        ex = jnp.exp(jnp.maximum(z, 0.01 * z))
        return rl, ilc, m, ex

    def p1_append(vals, cnt):
        rl, ilc, m, ex = vals
        plsc.addupdate_scatter(den_v, [ilc], ex, mask=m)
        pk = ilc | (rl << 10)
        scnt = lax.rem(cnt, STG)
        fb = cnt // STG
        plsc.store_compressed(spk_v.at[pl.ds(scnt, L)], pk, mask=m)
        plsc.store_compressed(sex_v.at[pl.ds(scnt, L)], ex, mask=m)
        cnt = cnt + plsc.all_reduce_population_count(m)[0]
        scnt2 = lax.rem(cnt, STG)

        @pl.when((cnt // STG > fb) & (fb < ABLKS))
        def _():
            pltpu.sync_copy(spk_v.at[pl.ds(0, STG)], apk.at[wid, fb])
            pltpu.sync_copy(sex_v.at[pl.ds(0, STG)], aex.at[wid, fb])
            rem_pk = spk_v[pl.ds(STG, L)]
            rem_ex = sex_v[pl.ds(STG, L)]
            rmask = lax.iota(jnp.int32, L) < scnt2
            plsc.store_compressed(spk_v.at[pl.ds(0, L)], rem_pk, mask=rmask)
            plsc.store_compressed(sex_v.at[pl.ds(0, L)], rem_ex, mask=rmask)
        return jnp.minimum(cnt, ACAP - 1)

    issue_edges(0, 0)

    def p1_block(b, cnt):
        blk = lax.rem(b, 2)
        wait_edges(blk)

        @pl.when(b + 1 < NBLK)
        def _():
            issue_edges(b + 1, 1 - blk)

        def step(i, cnt):
            vals = [edge_vec(blk, i * 4 + u) for u in range(4)]
            for u in range(4):
                cnt = p1_append(vals[u], cnt)
            return cnt
        return lax.fori_loop(0, VPB // 4, step, cnt)
    cnt = lax.fori_loop(0, NBLK, p1_block, 0)

    lastfb = jnp.minimum(cnt // STG, ABLKS - 1)
    pltpu.sync_copy(spk_v.at[pl.ds(0, STG)], apk.at[wid, lastfb])
    pltpu.sync_copy(sex_v.at[pl.ds(0, STG)], aex.at[wid, lastfb])

    def issue_flush(p):
        pltpu.async_copy(xr_sh.at[br_v.at[p]], rows_v.at[p], sem)

    def wait_rows(p):
        pltpu.make_async_copy(x_r.at[pl.ds(0, BATCH)], rows_v.at[p],
                              sem).wait()

    def accum(p):
        def fgroup(g, carry):
            avec = ba_v[p, pl.ds(g * L, L)]
            ivec = bi_v[p, pl.ds(g * L, L)]
            for j in range(L):
                av = jnp.full((L,), avec[j])
                ii = ivec[j]
                for q in range(RH // L):
                    plsc.addupdate(acc_v.at[ii, pl.ds(q * L, L)],
                                   av * rows_v[p, g * L + j, pl.ds(q * L, L)])
            return carry
        lax.fori_loop(0, GRP, fgroup, 0)

    nblk2 = (cnt + STG - 1) // STG

    def p2_block(fb, carry):
        par, pend = carry
        pltpu.sync_copy(apk.at[wid, fb], rpk_v)
        pltpu.sync_copy(aex.at[wid, fb], rex_v)
        gbase0 = fb * STG

        def p2_batch(kb, carry):
            par, pend = carry
            for u in range(GRP):
                off = kb * BATCH + u * L
                pk = rpk_v[pl.ds(off, L)]
                exv = rex_v[pl.ds(off, L)]
                valid = (gbase0 + off + lax.iota(jnp.int32, L)) < cnt
                ilc = jnp.minimum(pk & 0x3FF, NPT - 1)
                rl = jnp.minimum((pk >> 10) & 0x3FF, R - 1)
                dv = plsc.load_gather(den_v, [ilc])
                a = jnp.where(valid, exv / (dv + 1e-16), 0.0)
                br_v[par, pl.ds(u * L, L)] = rl
                bi_v[par, pl.ds(u * L, L)] = ilc
                ba_v[par, pl.ds(u * L, L)] = a

            @pl.when(pend == 1)
            def _():
                wait_rows(1 - par)
                accum(1 - par)
            issue_flush(par)
            return (1 - par, pend * 0 + 1)
        return lax.fori_loop(0, STG // BATCH, p2_batch, (par, pend))
    par, pend = lax.fori_loop(0, nblk2, p2_block, (0, 0))

    @pl.when(pend == 1)
    def _():
        wait_rows(1 - par)
        accum(1 - par)

    pltpu.sync_copy(acc_v, out.at[wid])


def kernel(x_e, x_r, edge_index, rel, line_graph_index, line_graph_val,
           w_h, w_t, w_r):
    del line_graph_index, line_graph_val
    w_e = jnp.zeros((EH, 128), jnp.float32).at[:, 0].set(w_h).at[:, 1].set(w_t)
    w_rp = jnp.zeros((RH, 128), jnp.float32).at[:, 0].set(w_r)
    se, sr = _projections(x_e, x_r, w_e, w_rp)
    s_ht = se[:, :2].T.reshape(-1)
    s_r1 = sr[:, 0]
    eflat = edge_index.reshape(-1)
    o = _sc_gat(s_ht, s_r1, eflat, rel, x_r)[0]
    x_e_h = o[:NS].reshape(N, RH)
    x_e_t = o[NS:].reshape(N, RH)
    return jnp.concatenate([x_e_h, x_e_t], axis=1)


# speedup vs baseline: 1.4039x; 1.0143x over previous
import functools

import jax
import jax.numpy as jnp
from jax import lax
from jax.experimental import pallas as pl
from jax.experimental.pallas import tpu as pltpu
from jax.experimental.pallas import tpu_sc as plsc

N = 10000
E = 320000
R = 1000
EH = 128
RH = 128

NC = 2
NS = 16
L = 16

NPT = N // NS
EBLK = 2000
NBLK = E // EBLK
VPB = EBLK // L
GRP = 4
BATCH = GRP * L
STG = 1024
ABLKS = 25
ACAP = ABLKS * STG


def _proj_body(xe_ref, xr_ref, we_ref, wr_ref, se_ref, sr_ref):
    se_ref[...] = jnp.dot(xe_ref[...], we_ref[...],
                          preferred_element_type=jnp.float32)
    sr_ref[...] = jnp.dot(xr_ref[...], wr_ref[...],
                          preferred_element_type=jnp.float32)


def _projections(x_e, x_r, w_e, w_r):
    return pl.pallas_call(
        _proj_body,
        out_shape=(jax.ShapeDtypeStruct((N, 128), jnp.float32),
                   jax.ShapeDtypeStruct((R, 128), jnp.float32)),
    )(x_e, x_r, w_e, w_r)


_mesh = plsc.VectorSubcoreMesh(core_axis_name="c", subcore_axis_name="s")


@functools.partial(
    pl.kernel,
    out_type=(jax.ShapeDtypeStruct((NC * NS, NPT, RH), jnp.float32),
              jax.ShapeDtypeStruct((NC * NS, ABLKS, STG), jnp.int32),
              jax.ShapeDtypeStruct((NC * NS, ABLKS, STG), jnp.float32)),
    mesh=_mesh,
    scratch_types=[
        pltpu.VMEM((N,), jnp.float32),
        pltpu.VMEM((R,), jnp.float32),
        pltpu.VMEM((2 * EBLK,), jnp.int32),
        pltpu.VMEM((2 * EBLK,), jnp.int32),
        pltpu.VMEM((NPT + 16,), jnp.float32),
        pltpu.VMEM((NPT, RH), jnp.float32),
        pltpu.VMEM((STG + L,), jnp.int32),
        pltpu.VMEM((STG + L,), jnp.float32),
        pltpu.VMEM((STG,), jnp.int32),
        pltpu.VMEM((STG,), jnp.float32),
        pltpu.VMEM((2, BATCH), jnp.int32),
        pltpu.VMEM((2, BATCH), jnp.int32),
        pltpu.VMEM((2, BATCH), jnp.float32),
        pltpu.VMEM((2, BATCH, RH), jnp.float32),
        pltpu.VMEM_SHARED((R, RH), jnp.float32),
        pltpu.SemaphoreType.DMA,
        pltpu.SemaphoreType.DMA,
    ],
    compiler_params=pltpu.CompilerParams(needs_layout_passes=False),
)
def _sc_gat(s_ht, s_r, eidx, rel, x_r, out, apk, aex,
            s_v, sr_v, idx_v, rel_v, den_v, acc_v,
            spk_v, sex_v, rpk_v, rex_v,
            br_v, bi_v, ba_v, rows_v, xr_sh, sem, sem_e):
    c = lax.axis_index("c")
    s = lax.axis_index("s")
    base = s * NPT
    wid = c * NS + s

    @pl.when(s == 0)
    def _():
        pltpu.sync_copy(x_r, xr_sh)

    pltpu.sync_copy(s_ht.at[pl.ds(c * N, N)], s_v)
    pltpu.sync_copy(s_r, sr_v)

    zf = jnp.zeros((L,), jnp.float32)

    def zden(i, carry):
        den_v[pl.ds(i * L, L)] = zf
        return carry
    lax.fori_loop(0, (NPT + 16) // L, zden, 0)

    def zacc(i, carry):
        for j in range(RH // L):
            acc_v[i, pl.ds(j * L, L)] = zf
        return carry
    lax.fori_loop(0, NPT, zacc, 0)

    plsc.subcore_barrier()

    def issue_edges(b, blk):
        pltpu.async_copy(eidx.at[pl.ds(c * E + b * EBLK, EBLK)],
                         idx_v.at[pl.ds(blk * EBLK, EBLK)], sem_e)
        pltpu.async_copy(rel.at[pl.ds(b * EBLK, EBLK)],
                         rel_v.at[pl.ds(blk * EBLK, EBLK)], sem_e)

    def wait_edges(blk):
        pltpu.make_async_copy(rel.at[pl.ds(0, EBLK)],
                              idx_v.at[pl.ds(blk * EBLK, EBLK)], sem_e).wait()
        pltpu.make_async_copy(rel.at[pl.ds(0, EBLK)],
                              rel_v.at[pl.ds(blk * EBLK, EBLK)], sem_e).wait()

    def edge_vec(blk, i):
        nd = idx_v[pl.ds(blk * EBLK + i * L, L)]
        rl = rel_v[pl.ds(blk * EBLK + i * L, L)]
        il = nd - base
        m = (il >= 0) & (il < NPT)
        ilc = jnp.clip(il, 0, NPT - 1)
        sh = plsc.load_gather(s_v, [nd])
        sr = plsc.load_gather(sr_v, [rl])
        z = sh + sr
        ex = jnp.exp(jnp.maximum(z, 0.01 * z))
        return rl, ilc, m, ex

    def p1_append(vals, cnt):
        rl, ilc, m, ex = vals
        plsc.addupdate_scatter(den_v, [ilc], ex, mask=m)
        pk = ilc | (rl << 10)
        scnt = lax.rem(cnt, STG)
        fb = cnt // STG
        plsc.store_compressed(spk_v.at[pl.ds(scnt, L)], pk, mask=m)
        plsc.store_compressed(sex_v.at[pl.ds(scnt, L)], ex, mask=m)
        cnt = cnt + plsc.all_reduce_population_count(m)[0]
        scnt2 = lax.rem(cnt, STG)

        @pl.when((cnt // STG > fb) & (fb < ABLKS))
        def _():
            pltpu.sync_copy(spk_v.at[pl.ds(0, STG)], apk.at[wid, fb])
            pltpu.sync_copy(sex_v.at[pl.ds(0, STG)], aex.at[wid, fb])
            rem_pk = spk_v[pl.ds(STG, L)]
            rem_ex = sex_v[pl.ds(STG, L)]
            rmask = lax.iota(jnp.int32, L) < scnt2
            plsc.store_compressed(spk_v.at[pl.ds(0, L)], rem_pk, mask=rmask)
            plsc.store_compressed(sex_v.at[pl.ds(0, L)], rem_ex, mask=rmask)
        return jnp.minimum(cnt, ACAP - 1)

    issue_edges(0, 0)

    def p1_block(b, cnt):
        blk = lax.rem(b, 2)
        wait_edges(blk)

        @pl.when(b + 1 < NBLK)
        def _():
            issue_edges(b + 1, 1 - blk)

        def step(i, cnt):
            vals = [edge_vec(blk, i * 5 + u) for u in range(5)]
            for u in range(5):
                cnt = p1_append(vals[u], cnt)
            return cnt
        return lax.fori_loop(0, VPB // 5, step, cnt)
    cnt = lax.fori_loop(0, NBLK, p1_block, 0)

    lastfb = jnp.minimum(cnt // STG, ABLKS - 1)
    pltpu.sync_copy(spk_v.at[pl.ds(0, STG)], apk.at[wid, lastfb])
    pltpu.sync_copy(sex_v.at[pl.ds(0, STG)], aex.at[wid, lastfb])

    def issue_flush(p):
        pltpu.async_copy(xr_sh.at[br_v.at[p]], rows_v.at[p], sem)

    def wait_rows(p):
        pltpu.make_async_copy(x_r.at[pl.ds(0, BATCH)], rows_v.at[p],
                              sem).wait()

    def accum(p):
        def fgroup(g, carry):
            avec = ba_v[p, pl.ds(g * L, L)]
            ivec = bi_v[p, pl.ds(g * L, L)]
            for j in range(L):
                av = jnp.full((L,), avec[j])
                ii = ivec[j]
                for q in range(RH // L):
                    plsc.addupdate(acc_v.at[ii, pl.ds(q * L, L)],
                                   av * rows_v[p, g * L + j, pl.ds(q * L, L)])
            return carry
        lax.fori_loop(0, GRP, fgroup, 0)

    nblk2 = (cnt + STG - 1) // STG

    def p2_block(fb, carry):
        par, pend = carry
        pltpu.sync_copy(apk.at[wid, fb], rpk_v)
        pltpu.sync_copy(aex.at[wid, fb], rex_v)
        gbase0 = fb * STG

        def p2_batch(kb, carry):
            par, pend = carry
            for u in range(GRP):
                off = kb * BATCH + u * L
                pk = rpk_v[pl.ds(off, L)]
                exv = rex_v[pl.ds(off, L)]
                valid = (gbase0 + off + lax.iota(jnp.int32, L)) < cnt
                ilc = jnp.minimum(pk & 0x3FF, NPT - 1)
                rl = jnp.minimum((pk >> 10) & 0x3FF, R - 1)
                dv = plsc.load_gather(den_v, [ilc])
                a = jnp.where(valid, exv / (dv + 1e-16), 0.0)
                br_v[par, pl.ds(u * L, L)] = rl
                bi_v[par, pl.ds(u * L, L)] = ilc
                ba_v[par, pl.ds(u * L, L)] = a

            @pl.when(pend == 1)
            def _():
                wait_rows(1 - par)
                accum(1 - par)
            issue_flush(par)
            return (1 - par, pend * 0 + 1)
        return lax.fori_loop(0, STG // BATCH, p2_batch, (par, pend))
    par, pend = lax.fori_loop(0, nblk2, p2_block, (0, 0))

    @pl.when(pend == 1)
    def _():
        wait_rows(1 - par)
        accum(1 - par)

    pltpu.sync_copy(acc_v, out.at[wid])


def kernel(x_e, x_r, edge_index, rel, line_graph_index, line_graph_val,
           w_h, w_t, w_r):
    del line_graph_index, line_graph_val
    w_e = jnp.zeros((EH, 128), jnp.float32).at[:, 0].set(w_h).at[:, 1].set(w_t)
    w_rp = jnp.zeros((RH, 128), jnp.float32).at[:, 0].set(w_r)
    se, sr = _projections(x_e, x_r, w_e, w_rp)
    s_ht = se[:, :2].T.reshape(-1)
    s_r1 = sr[:, 0]
    eflat = edge_index.reshape(-1)
    o = _sc_gat(s_ht, s_r1, eflat, rel, x_r)[0]
    x_e_h = o[:NS].reshape(N, RH)
    x_e_t = o[NS:].reshape(N, RH)
    return jnp.concatenate([x_e_h, x_e_t], axis=1)
